# baseline (speedup 1.0000x reference)
import jax
import jax.numpy as jnp
from jax.experimental import pallas as pl


_STEP = 2


def _l2_min_body(ze_ref, emb_ref, out_ref):
    i = pl.program_id(0)
    e = emb_ref[...]
    en = (e * -2.0).astype(jnp.bfloat16)
    ee = jnp.sum(e * e, axis=1, keepdims=True)
    for j in range(_STEP):
        z = ze_ref[j]
        dot = jax.lax.dot_general(
            en, z.astype(jnp.bfloat16), (((1,), (0,)), ((), ())),
            preferred_element_type=jnp.float32,
        )
        zz = jnp.sum(z * z, axis=0)
        out_ref[i * _STEP + j, :] = jnp.min(dot + ee, axis=0) + zz


def kernel(ze, emb):
    B, Q, N = ze.shape
    K, _ = emb.shape
    return pl.pallas_call(
        _l2_min_body,
        grid=(B // _STEP,),
        in_specs=[
            pl.BlockSpec((_STEP, Q, N), lambda i: (i, 0, 0)),
            pl.BlockSpec((K, Q), lambda i: (0, 0)),
        ],
        out_specs=pl.BlockSpec((B, N), lambda i: (0, 0)),
        out_shape=jax.ShapeDtypeStruct((B, N), jnp.float32),
    )(ze, emb)
